# baseline (speedup 1.0000x reference)
import functools

import jax
import jax.numpy as jnp
from jax import lax
from jax.experimental import pallas as pl
from jax.experimental.pallas import tpu as pltpu
from jax.experimental.pallas import tpu_sc as plsc

_LENGTH = 8192
_CHANNELS = 1024
_SEQ = 8192
_BATCH = 4

_B = _SEQ * _BATCH

_NC = 2
_NS = 16
_NW = _NC * _NS
_C = 32
_D = 3

_N_SC = 20480
_BPW = _N_SC // _NW
_G = _BPW // _C

_N_TC = _B - _N_SC
_K = 8

_mesh = plsc.VectorSubcoreMesh(core_axis_name="c", subcore_axis_name="s")


@functools.partial(
    pl.kernel,
    out_type=jax.ShapeDtypeStruct((_N_SC, _CHANNELS), jnp.float32),
    mesh=_mesh,
    scratch_types=[
        pltpu.VMEM((_G, _C), jnp.int32),
        pltpu.VMEM((_D, _C, _CHANNELS), jnp.float32),
        pltpu.SemaphoreType.DMA,
        pltpu.SemaphoreType.DMA,
        pltpu.SemaphoreType.DMA,
        pltpu.SemaphoreType.DMA,
        pltpu.SemaphoreType.DMA,
        pltpu.SemaphoreType.DMA,
    ],
)
def _sc_gather(idx_hbm, table_hbm, out_hbm, idx_v, buf,
               gs0, gs1, gs2, ss0, ss1, ss2):
    gsems = [gs0, gs1, gs2]
    ssems = [ss0, ss1, ss2]
    wid = lax.axis_index("s") * _NC + lax.axis_index("c")
    base = wid * _BPW
    pltpu.sync_copy(idx_hbm.at[wid], idx_v)

    def start_gather(g):
        b = g % _D
        pltpu.async_copy(table_hbm.at[idx_v.at[g]], buf.at[b], gsems[b])

    def wait_gather(g):
        b = g % _D
        pltpu.make_async_copy(table_hbm.at[idx_v.at[g]], buf.at[b],
                              gsems[b]).wait()

    def start_scatter(g):
        b = g % _D
        pltpu.async_copy(buf.at[b], out_hbm.at[pl.ds(base + g * _C, _C)],
                         ssems[b])

    def wait_scatter(g):
        b = g % _D
        pltpu.make_async_copy(buf.at[b],
                              out_hbm.at[pl.ds(base + g * _C, _C)],
                              ssems[b]).wait()

    start_gather(0)
    start_gather(1)
    for g in range(_G):
        wait_gather(g)
        start_scatter(g)
        if g >= 1:
            wait_scatter(g - 1)
        if g + 2 < _G:
            start_gather(g + 2)
    wait_scatter(_G - 1)


_SUB = 8
_LANE = _CHANNELS // _SUB


def _tc_imap(k):
    def imap(b, idx_ref):
        return (idx_ref[_K * b + k], 0, 0)
    return imap


def _tc_body(idx_ref, *refs):
    ins = refs[:_K]
    out = refs[_K]
    for k in range(_K):
        out[k, :, :] = ins[k][0, :, :]


_tc_grid_spec = pltpu.PrefetchScalarGridSpec(
    num_scalar_prefetch=1,
    grid=(_N_TC // _K,),
    in_specs=[pl.BlockSpec((1, _SUB, _LANE), _tc_imap(k)) for k in range(_K)],
    out_specs=pl.BlockSpec((_K, _SUB, _LANE), lambda b, idx_ref: (b, 0, 0)),
)

_tc_gather = pl.pallas_call(
    _tc_body,
    grid_spec=_tc_grid_spec,
    out_shape=jax.ShapeDtypeStruct((_N_TC, _SUB, _LANE), jnp.float32),
)


def kernel(i, encoding):
    idx = i.astype(jnp.int32).reshape(-1)
    idx_sc = idx[:_N_SC].reshape(_NW, _G, _C)
    idx_tc = idx[_N_SC:]
    out_sc = _sc_gather(idx_sc, encoding)
    enc3 = encoding.reshape(_LENGTH, _SUB, _LANE)
    out_tc = _tc_gather(idx_tc, *([enc3] * _K)).reshape(_N_TC, _CHANNELS)
    out = jnp.concatenate([out_sc, out_tc], axis=0)
    return out.reshape(_SEQ, _BATCH, _CHANNELS)


# speedup vs baseline: 4.3844x; 4.3844x over previous
import functools

import jax
import jax.numpy as jnp
from jax import lax
from jax.experimental import pallas as pl
from jax.experimental.pallas import tpu as pltpu
from jax.experimental.pallas import tpu_sc as plsc

_LENGTH = 8192
_CHANNELS = 1024
_SEQ = 8192
_BATCH = 4

_NC = 2
_NS = 16
_NW = _NC * _NS
_B = _SEQ * _BATCH
_BPW = _B // _NW
_C = 16
_G = _BPW // _C
_D = 6
_A = 3

_mesh = plsc.VectorSubcoreMesh(core_axis_name="c", subcore_axis_name="s")


@functools.partial(
    pl.kernel,
    out_type=jax.ShapeDtypeStruct((_B, _CHANNELS), jnp.float32),
    mesh=_mesh,
    scratch_types=[
        pltpu.VMEM((_G, _C), jnp.int32),
        pltpu.VMEM((_D, _C, _CHANNELS), jnp.float32),
    ] + [pltpu.SemaphoreType.DMA] * (2 * _D),
)
def _sc_gather(idx_hbm, table_hbm, out_hbm, idx_v, buf, *sems):
    gsems = sems[:_D]
    ssems = sems[_D:]
    wid = lax.axis_index("s") * _NC + lax.axis_index("c")
    base = wid * _BPW
    pltpu.sync_copy(idx_hbm.at[wid], idx_v)

    def start_gather(g):
        b = g % _D
        pltpu.async_copy(table_hbm.at[idx_v.at[g]], buf.at[b], gsems[b])

    def wait_gather(g):
        b = g % _D
        pltpu.make_async_copy(table_hbm.at[idx_v.at[g]], buf.at[b],
                              gsems[b]).wait()

    def start_scatter(g):
        b = g % _D
        pltpu.async_copy(buf.at[b], out_hbm.at[pl.ds(base + g * _C, _C)],
                         ssems[b])

    def wait_scatter(g):
        b = g % _D
        pltpu.make_async_copy(buf.at[b],
                              out_hbm.at[pl.ds(base + g * _C, _C)],
                              ssems[b]).wait()

    for g in range(_A):
        start_gather(g)
    for g in range(_G):
        wait_gather(g)
        start_scatter(g)
        if g >= _A:
            wait_scatter(g - _A)
        if g + _A < _G:
            start_gather(g + _A)
    for g in range(_G - _A, _G):
        wait_scatter(g)


def kernel(i, encoding):
    idx = i.astype(jnp.int32).reshape(_NW, _G, _C)
    out = _sc_gather(idx, encoding)
    return out.reshape(_SEQ, _BATCH, _CHANNELS)
